# Initial kernel scaffold; baseline (speedup 1.0000x reference)
import jax
import jax.numpy as jnp
from jax.experimental import pallas as pl


def kernel(h, node_feat, W, b):
    raise NotImplementedError("write your pallas kernel here")


# speedup vs baseline: 1.1078x; 1.1078x over previous
import jax
import jax.numpy as jnp
from jax.experimental import pallas as pl

N = 10000
DEG = 32
D = 128
OUT = 128
BLK = 400


def _tc_body(h_ref, nf_ref, w1t_ref, w2t_ref, b_ref, out_ref):
    hm = jnp.mean(h_ref[...], axis=1)
    out_ref[...] = (
        jnp.dot(hm, w1t_ref[...], preferred_element_type=jnp.float32)
        + jnp.dot(nf_ref[...], w2t_ref[...], preferred_element_type=jnp.float32)
        + b_ref[...]
    )


def kernel(h, node_feat, W, b):
    w1t = W[:, :D].T
    w2t = W[:, D:].T
    b2 = b.reshape(1, OUT)
    grid = N // BLK
    return pl.pallas_call(
        _tc_body,
        grid=(grid,),
        in_specs=[
            pl.BlockSpec((BLK, DEG, D), lambda i: (i, 0, 0)),
            pl.BlockSpec((BLK, D), lambda i: (i, 0)),
            pl.BlockSpec((D, OUT), lambda i: (0, 0)),
            pl.BlockSpec((D, OUT), lambda i: (0, 0)),
            pl.BlockSpec((1, OUT), lambda i: (0, 0)),
        ],
        out_specs=pl.BlockSpec((BLK, OUT), lambda i: (i, 0)),
        out_shape=jax.ShapeDtypeStruct((N, OUT), jnp.float32),
    )(h, node_feat, w1t, w2t, b2)
